# Initial kernel scaffold; baseline (speedup 1.0000x reference)
import jax
import jax.numpy as jnp
from jax.experimental import pallas as pl


def kernel(h_edge, khop_edge_index_0, khop_edge_index_1, gate, W0, b0, W1, b1):
    raise NotImplementedError("write your pallas kernel here")


# speedup vs baseline: 12.2468x; 12.2468x over previous
import functools

import jax
import jax.numpy as jnp
from jax import lax
from jax.experimental import pallas as pl
from jax.experimental.pallas import tpu as pltpu
from jax.experimental.pallas import tpu_sc as plsc

_N = 10000
_D = 256
_E = 160000
_H = 128
_NTILE = 16
_CHUNK = 128
_NCHUNK = 79
_EPT = _NCHUNK * _CHUNK
_EP = _NTILE * _EPT
_AVEC = _EPT // 16
_CROW = 640
_NPAD = 10240
_ZROWS = _NPAD
_STRIPE = _NPAD // _NTILE
_RBLK = 1024


def _sc_mesh():
    return plsc.VectorSubcoreMesh(core_axis_name="c", subcore_axis_name="s")


_SC_PARAMS = pltpu.CompilerParams(needs_layout_passes=False)


def _deg_body(dst_hbm, out_hbm, idx_v, cnt_v):
    c = lax.axis_index("c")
    s = lax.axis_index("s")
    pltpu.sync_copy(dst_hbm.at[c, s], idx_v)
    zeros = jnp.zeros((16,), jnp.float32)

    def zloop(j, car):
        cnt_v[pl.ds(j * 16, 16)] = zeros
        return car

    lax.fori_loop(0, _NPAD // 16, zloop, 0)
    ones = jnp.ones((16,), jnp.float32)

    def aloop(j, car):
        plsc.addupdate_scatter(cnt_v, [idx_v[pl.ds(j * 16, 16)]], ones)
        return car

    lax.fori_loop(0, _EPT // 16, aloop, 0)
    pltpu.sync_copy(cnt_v, out_hbm.at[c, s])


def _run_deg(dst_a):
    k = functools.partial(
        pl.kernel,
        out_type=jax.ShapeDtypeStruct((2, _NTILE, _NPAD), jnp.float32),
        mesh=_sc_mesh(),
        scratch_types=[
            pltpu.VMEM((_EPT,), jnp.int32),
            pltpu.VMEM((_NPAD,), jnp.float32),
        ],
        compiler_params=_SC_PARAMS,
    )(_deg_body)
    return k(dst_a)


def _scatter_body(y0l, y0r, y1l, y1r, s0, d0, s1, d1,
                  z0l, z0r, z1l, z1r, srcv, dstv, rows, zsp, sem):
    c = lax.axis_index("c")
    s = lax.axis_index("s")
    base = s * _STRIPE

    def run(y, z):
        pltpu.sync_copy(y.at[pl.ds(base, _STRIPE)], zsp.at[pl.ds(base, _STRIPE)])
        plsc.subcore_barrier()

        def chunk(j, car):
            pltpu.async_copy(y.at[srcv.at[j]], rows, sem).wait()
            pltpu.sync_copy(rows, zsp.at[dstv.at[j]], add=True)
            return car

        lax.fori_loop(0, _NCHUNK, chunk, 0)
        plsc.subcore_barrier()
        pltpu.sync_copy(zsp.at[pl.ds(base, _STRIPE)], z.at[pl.ds(base, _STRIPE)])
        plsc.subcore_barrier()

    for (s_r, d_r, yl, yr, zl, zr) in (
        (s0, d0, y0l, y0r, z0l, z0r),
        (s1, d1, y1l, y1r, z1l, z1r),
    ):
        pltpu.sync_copy(s_r.at[s], srcv)
        pltpu.sync_copy(d_r.at[s], dstv)
        pl.when(c == 0)(functools.partial(run, yl, zl))
        pl.when(c == 1)(functools.partial(run, yr, zr))


def _run_scatter(y0l, y0r, y1l, y1r, s0, d0, s1, d1):
    zt = jax.ShapeDtypeStruct((_NPAD, _H), jnp.float32)
    k = functools.partial(
        pl.kernel,
        out_type=(zt, zt, zt, zt),
        mesh=_sc_mesh(),
        scratch_types=[
            pltpu.VMEM((_NCHUNK, _CHUNK), jnp.int32),
            pltpu.VMEM((_NCHUNK, _CHUNK), jnp.int32),
            pltpu.VMEM((_CHUNK, _H), jnp.float32),
            pltpu.VMEM_SHARED((_ZROWS, _H), jnp.float32),
            pltpu.SemaphoreType.DMA,
        ],
        compiler_params=_SC_PARAMS,
    )(_scatter_body)
    return k(y0l, y0r, y1l, y1r, s0, d0, s1, d1)


def _xw_body(h_ref, w0_ref, w1_ref, cnt_ref,
             y0l_ref, y0r_ref, y1l_ref, y1r_ref):
    deg = jnp.sum(cnt_ref[...], axis=1) + 1.0
    dis = lax.rsqrt(deg)
    h = h_ref[...]
    for k, (w_ref, yl, yr) in enumerate(
        ((w0_ref, y0l_ref, y0r_ref), (w1_ref, y1l_ref, y1r_ref))
    ):
        y = jnp.dot(h, w_ref[...], preferred_element_type=jnp.float32)
        y = y * dis[k][:, None]
        yl[...] = y[:, :_H]
        yr[...] = y[:, _H:]


def _run_xw(h_p, w0, w1, cnts):
    grid = _NPAD // _RBLK
    yt = jax.ShapeDtypeStruct((_NPAD, _H), jnp.float32)
    return pl.pallas_call(
        _xw_body,
        grid=(grid,),
        in_specs=[
            pl.BlockSpec((_RBLK, _D), lambda i: (i, 0)),
            pl.BlockSpec((_D, _D), lambda i: (0, 0)),
            pl.BlockSpec((_D, _D), lambda i: (0, 0)),
            pl.BlockSpec((2, _NTILE, _RBLK), lambda i: (0, 0, i)),
        ],
        out_specs=[pl.BlockSpec((_RBLK, _H), lambda i: (i, 0))] * 4,
        out_shape=[yt] * 4,
    )(h_p, w0, w1, cnts)


def _combine_body(z0l_ref, z0r_ref, z1l_ref, z1r_ref, cnt_ref, gate_ref,
                  b0_ref, b1_ref, out_ref):
    deg = jnp.sum(cnt_ref[...], axis=1) + 1.0
    dis = lax.rsqrt(deg)
    g = gate_ref[...]
    z0 = jnp.concatenate([z0l_ref[...], z0r_ref[...]], axis=1)
    z1 = jnp.concatenate([z1l_ref[...], z1r_ref[...]], axis=1)
    h0 = jnp.maximum(dis[0][:, None] * z0 + b0_ref[...], 0.0)
    h1 = jnp.maximum(dis[1][:, None] * z1 + b1_ref[...], 0.0)
    out_ref[...] = g[:, 0:1] * h0 + g[:, 1:2] * h1


def _run_combine(z0l, z0r, z1l, z1r, cnts, gate_p, b0, b1):
    grid = _NPAD // _RBLK
    return pl.pallas_call(
        _combine_body,
        grid=(grid,),
        in_specs=[
            pl.BlockSpec((_RBLK, _H), lambda i: (i, 0)),
            pl.BlockSpec((_RBLK, _H), lambda i: (i, 0)),
            pl.BlockSpec((_RBLK, _H), lambda i: (i, 0)),
            pl.BlockSpec((_RBLK, _H), lambda i: (i, 0)),
            pl.BlockSpec((2, _NTILE, _RBLK), lambda i: (0, 0, i)),
            pl.BlockSpec((_RBLK, 2), lambda i: (i, 0)),
            pl.BlockSpec((1, _D), lambda i: (0, 0)),
            pl.BlockSpec((1, _D), lambda i: (0, 0)),
        ],
        out_specs=pl.BlockSpec((_RBLK, _D), lambda i: (i, 0)),
        out_shape=jax.ShapeDtypeStruct((_NPAD, _D), jnp.float32),
    )(z0l, z0r, z1l, z1r, cnts, gate_p, b0, b1)


def kernel(h_edge, khop_edge_index_0, khop_edge_index_1, gate, W0, b0, W1, b1):
    npad = _EP - _E
    src_pad = jnp.zeros((npad,), jnp.int32)
    dst_pad = jnp.full((npad,), _N, jnp.int32)

    def prep(ei):
        src = jnp.concatenate([ei[0].astype(jnp.int32), src_pad])
        dst = jnp.concatenate([ei[1].astype(jnp.int32), dst_pad])
        return src, dst

    src0, dst0 = prep(khop_edge_index_0)
    src1, dst1 = prep(khop_edge_index_1)

    dst_a = jnp.stack([dst0.reshape(_NTILE, _EPT),
                       dst1.reshape(_NTILE, _EPT)])
    cnts = _run_deg(dst_a)

    h_p = jnp.pad(h_edge, ((0, _NPAD - _N), (0, 0)))
    y0l, y0r, y1l, y1r = _run_xw(h_p, W0, W1, cnts)

    s0r = src0.reshape(_NTILE, _NCHUNK, _CHUNK)
    d0r = dst0.reshape(_NTILE, _NCHUNK, _CHUNK)
    s1r = src1.reshape(_NTILE, _NCHUNK, _CHUNK)
    d1r = dst1.reshape(_NTILE, _NCHUNK, _CHUNK)
    z0l, z0r, z1l, z1r = _run_scatter(y0l, y0r, y1l, y1r, s0r, d0r, s1r, d1r)

    gate_p = jnp.pad(gate, ((0, _NPAD - _N), (0, 0)))
    out = _run_combine(z0l, z0r, z1l, z1r, cnts, gate_p,
                       b0.reshape(1, _D), b1.reshape(1, _D))
    return out[:_N]
